# baseline (speedup 1.0000x reference)
import jax
import jax.numpy as jnp
from jax.experimental import pallas as pl
from jax.experimental.pallas import tpu as pltpu

_PROB = 0.5


def _body(x_ref, n_ref, o_ref):
    x = x_ref[...]
    o_ref[...] = jnp.where(n_ref[...] < _PROB, jnp.round(x), x)


def kernel(x, noise):
    B, C, H, W = x.shape
    xt = jnp.transpose(x, (0, 2, 3, 1))
    nt = jnp.transpose(noise, (0, 2, 3, 1))
    spec = pl.BlockSpec((1, H, W, C), lambda i: (i, 0, 0, 0))
    out = pl.pallas_call(
        _body,
        grid=(B,),
        in_specs=[spec, spec],
        out_specs=spec,
        out_shape=jax.ShapeDtypeStruct((B, H, W, C), x.dtype),
        compiler_params=pltpu.CompilerParams(
            dimension_semantics=("parallel",),
        ),
    )(xt, nt)
    return jnp.transpose(out, (0, 3, 1, 2))
